# baseline (speedup 1.0000x reference)
import functools

import jax
import jax.numpy as jnp
from jax import lax
from jax.experimental import pallas as pl
from jax.experimental.pallas import tpu as pltpu
from jax.experimental.pallas import tpu_sc as plsc

N = 2048
H = 768
E = 8
CAP = 1024
XS_ROWS = E * CAP
TRASH = XS_ROWS
XS_TOT = XS_ROWS + 8

NC, NS = 2, 16
NW = NC * NS
TOK_PER = N // NW
LANES = 16


ACH = 8
ACS = N // ACH


def _routing_body(tok_ref, wg_ref, d1_ref, d2_ref, wa_ref, wb_ref, cnt_ref,
                  s_sc, i1_sc, i2_sc, m1_sc, m2_sc):
    g = pl.program_id(0)
    x = tok_ref[...]
    wg = wg_ref[...]
    logits = lax.dot_general(x, wg, (((1,), (1,)), ((), ())),
                             preferred_element_type=jnp.float32)
    m = jnp.max(logits, axis=1, keepdims=True)
    ex = jnp.exp(logits - m)
    probs = ex / jnp.sum(ex, axis=1, keepdims=True)

    lane = lax.broadcasted_iota(jnp.int32, (ACS, E), 1)
    m1 = jnp.max(probs, axis=1, keepdims=True)
    i1 = jnp.min(jnp.where(probs == m1, lane, E), axis=1, keepdims=True)
    probs2 = jnp.where(lane == i1, -1.0, probs)
    m2 = jnp.max(probs2, axis=1, keepdims=True)
    i2 = jnp.min(jnp.where(probs2 == m2, lane, E), axis=1, keepdims=True)

    rows = pl.ds(g * ACS, ACS)
    s_sc[rows, :] = (lane == i1).astype(jnp.float32) \
        + (lane == i2).astype(jnp.float32)
    i1_sc[rows, :] = i1
    i2_sc[rows, :] = i2
    m1_sc[rows, :] = m1
    m2_sc[rows, :] = m2

    @pl.when(g == ACH - 1)
    def _():
        s = s_sc[...]
        kc = 256
        row_i = lax.broadcasted_iota(jnp.int32, (N, kc), 0)
        col_i = lax.broadcasted_iota(jnp.int32, (N, kc), 1)
        c = jnp.zeros((N, E), jnp.float32)
        for k in range(N // kc):
            mask = (col_i + k * kc < row_i).astype(jnp.float32)
            c = c + lax.dot_general(mask, s[k * kc:(k + 1) * kc, :],
                                    (((1,), (0,)), ((), ())),
                                    preferred_element_type=jnp.float32)

        counts = jnp.sum(s, axis=0, keepdims=True)
        counts_i = counts.astype(jnp.int32)
        kept = counts_i <= CAP

        lane_n = lax.broadcasted_iota(jnp.int32, (N, E), 1)
        i1a = i1_sc[...]
        i2a = i2_sc[...]
        oh1 = (lane_n == i1a).astype(jnp.float32)
        oh2 = (lane_n == i2a).astype(jnp.float32)
        r1 = jnp.sum(c * oh1, axis=1, keepdims=True)
        r2 = jnp.sum(c * oh2, axis=1, keepdims=True)
        keptf = kept.astype(jnp.float32)
        k1 = jnp.sum(keptf * oh1, axis=1, keepdims=True)
        k2 = jnp.sum(keptf * oh2, axis=1, keepdims=True)
        ro1 = i1a.astype(jnp.float32) * float(CAP)
        ro2 = i2a.astype(jnp.float32) * float(CAP)

        d1_ref[...] = jnp.where(k1 > 0., ro1 + r1,
                                float(TRASH)).astype(jnp.int32)
        d2_ref[...] = jnp.where(k2 > 0., ro2 + r2,
                                float(TRASH)).astype(jnp.int32)
        wa_ref[...] = jnp.broadcast_to(m1_sc[...] * k1, (N, LANES))
        wb_ref[...] = jnp.broadcast_to(m2_sc[...] * k2, (N, LANES))
        cnt_ref[...] = counts_i


def _routing(flat_tokens, w_gate):
    grid_spec = pltpu.PrefetchScalarGridSpec(
        num_scalar_prefetch=0,
        grid=(ACH,),
        in_specs=[
            pl.BlockSpec((ACS, H), lambda g: (g, 0)),
            pl.BlockSpec((E, H), lambda g: (0, 0)),
        ],
        out_specs=[
            pl.BlockSpec((N, 1), lambda g: (0, 0)),
            pl.BlockSpec((N, 1), lambda g: (0, 0)),
            pl.BlockSpec((N, LANES), lambda g: (0, 0)),
            pl.BlockSpec((N, LANES), lambda g: (0, 0)),
            pl.BlockSpec((1, E), lambda g: (0, 0)),
        ],
        scratch_shapes=[
            pltpu.VMEM((N, E), jnp.float32),
            pltpu.VMEM((N, 1), jnp.int32),
            pltpu.VMEM((N, 1), jnp.int32),
            pltpu.VMEM((N, 1), jnp.float32),
            pltpu.VMEM((N, 1), jnp.float32),
        ],
    )
    return pl.pallas_call(
        _routing_body,
        grid_spec=grid_spec,
        out_shape=[
            jax.ShapeDtypeStruct((N, 1), jnp.int32),
            jax.ShapeDtypeStruct((N, 1), jnp.int32),
            jax.ShapeDtypeStruct((N, LANES), jnp.float32),
            jax.ShapeDtypeStruct((N, LANES), jnp.float32),
            jax.ShapeDtypeStruct((1, E), jnp.int32),
        ],
        compiler_params=pltpu.CompilerParams(
            dimension_semantics=("arbitrary",)),
    )(flat_tokens, w_gate)


def _dispatch_body(tok_hbm, d1_hbm, d2_hbm, xs_hbm, d1_v, d2_v, rows_v,
                   sem_t, sem1, sem2):
    c = lax.axis_index("c")
    s = lax.axis_index("s")
    wid = s * NC + c
    base = wid * TOK_PER
    tok_cp = pltpu.async_copy(tok_hbm.at[pl.ds(base, TOK_PER)], rows_v, sem_t)
    pltpu.sync_copy(d1_hbm.at[pl.ds(base, TOK_PER)], d1_v)
    pltpu.sync_copy(d2_hbm.at[pl.ds(base, TOK_PER)], d2_v)
    tok_cp.wait()
    cp1 = pltpu.async_copy(rows_v, xs_hbm.at[d1_v], sem1)
    cp2 = pltpu.async_copy(rows_v, xs_hbm.at[d2_v], sem2)
    cp1.wait()
    cp2.wait()


def _dispatch(flat_tokens, d1, d2):
    mesh = plsc.VectorSubcoreMesh(core_axis_name="c", subcore_axis_name="s")
    return pl.kernel(
        _dispatch_body,
        out_type=jax.ShapeDtypeStruct((XS_TOT, H), jnp.float32),
        mesh=mesh,
        scratch_types=[
            pltpu.VMEM((TOK_PER,), jnp.int32),
            pltpu.VMEM((TOK_PER,), jnp.int32),
            pltpu.VMEM((TOK_PER, H), jnp.float32),
            pltpu.SemaphoreType.DMA,
            pltpu.SemaphoreType.DMA,
            pltpu.SemaphoreType.DMA,
        ],
    )(flat_tokens, d1, d2)


def _expert_body(cnt_ref, xs_ref, w1_ref, b1_ref, w2_ref, b2_ref, ys_ref):
    e = pl.program_id(0)
    cnt = cnt_ref[e]

    @pl.when((cnt > 0) & (cnt <= CAP))
    def _():
        x = xs_ref[...].astype(jnp.bfloat16)
        h = lax.dot_general(x, w1_ref[0].astype(jnp.bfloat16),
                            (((1,), (1,)), ((), ())),
                            preferred_element_type=jnp.float32)
        h = h + b1_ref[0]
        h = h * (1.0 / (1.0 + jnp.exp(-h)))
        y = lax.dot_general(h.astype(jnp.bfloat16),
                            w2_ref[0].astype(jnp.bfloat16),
                            (((1,), (1,)), ((), ())),
                            preferred_element_type=jnp.float32)
        ys_ref[...] = y + b2_ref[0]


def _experts(cnt, xs, w1, b1, w2, b2):
    def emap(e, cnt_s):
        return (e, 0, 0)

    grid_spec = pltpu.PrefetchScalarGridSpec(
        num_scalar_prefetch=1,
        grid=(E,),
        in_specs=[
            pl.BlockSpec((CAP, H), lambda e, cnt_s: (e, 0)),
            pl.BlockSpec((1, H, H), emap),
            pl.BlockSpec((1, 1, H), emap),
            pl.BlockSpec((1, H, H), emap),
            pl.BlockSpec((1, 1, H), emap),
        ],
        out_specs=pl.BlockSpec((CAP, H), lambda e, cnt_s: (e, 0)),
    )
    return pl.pallas_call(
        _expert_body,
        grid_spec=grid_spec,
        out_shape=jax.ShapeDtypeStruct((XS_ROWS, H), jnp.float32),
        compiler_params=pltpu.CompilerParams(
            dimension_semantics=("arbitrary",)),
    )(cnt, xs, w1, b1.reshape(E, 1, H), w2, b2.reshape(E, 1, H))


def _combine_body(ys_hbm, d1_hbm, d2_hbm, wa_hbm, wb_hbm, out_hbm,
                  d1_v, d2_v, wa_v, wb_v, r1_v, r2_v, sem1, sem2):
    c = lax.axis_index("c")
    s = lax.axis_index("s")
    wid = s * NC + c
    base = wid * TOK_PER
    pltpu.sync_copy(d1_hbm.at[pl.ds(base, TOK_PER)], d1_v)
    pltpu.sync_copy(d2_hbm.at[pl.ds(base, TOK_PER)], d2_v)
    pltpu.sync_copy(wa_hbm.at[pl.ds(base, TOK_PER)], wa_v)
    pltpu.sync_copy(wb_hbm.at[pl.ds(base, TOK_PER)], wb_v)
    for k in range(TOK_PER // LANES):
        sl = pl.ds(k * LANES, LANES)
        d1_v[sl] = jnp.minimum(d1_v[sl], XS_ROWS - 1)
        d2_v[sl] = jnp.minimum(d2_v[sl], XS_ROWS - 1)
    cp1 = pltpu.async_copy(ys_hbm.at[d1_v], r1_v, sem1)
    cp2 = pltpu.async_copy(ys_hbm.at[d2_v], r2_v, sem2)
    cp1.wait()
    cp2.wait()

    def row(j, _):
        wa = wa_v[j, :]
        wb = wb_v[j, :]
        zero = jnp.zeros((LANES,), jnp.float32)
        for ch in range(H // LANES):
            sl = pl.ds(ch * LANES, LANES)
            a = r1_v[j, sl]
            b = r2_v[j, sl]
            r1_v[j, sl] = (jnp.where(wa == 0.0, zero, a * wa)
                           + jnp.where(wb == 0.0, zero, b * wb))
        return 0

    lax.fori_loop(0, TOK_PER, row, 0)
    pltpu.sync_copy(r1_v, out_hbm.at[pl.ds(base, TOK_PER)])


def _combine(ys, d1, d2, wa, wb):
    mesh = plsc.VectorSubcoreMesh(core_axis_name="c", subcore_axis_name="s")
    return pl.kernel(
        _combine_body,
        out_type=jax.ShapeDtypeStruct((N, H), jnp.float32),
        mesh=mesh,
        scratch_types=[
            pltpu.VMEM((TOK_PER,), jnp.int32),
            pltpu.VMEM((TOK_PER,), jnp.int32),
            pltpu.VMEM((TOK_PER, LANES), jnp.float32),
            pltpu.VMEM((TOK_PER, LANES), jnp.float32),
            pltpu.VMEM((TOK_PER, H), jnp.float32),
            pltpu.VMEM((TOK_PER, H), jnp.float32),
            pltpu.SemaphoreType.DMA,
            pltpu.SemaphoreType.DMA,
        ],
    )(ys, d1, d2, wa, wb)


def kernel(tokens, W_gate, W1, b1, W2, b2):
    batch, seq, hidden = tokens.shape
    flat = tokens.reshape(batch * seq, hidden)
    d1, d2, wa, wb, cnt = _routing(flat, W_gate)
    d1 = d1.reshape(N)
    d2 = d2.reshape(N)
    xs = _dispatch(flat, d1, d2)
    ys = _experts(cnt.reshape(E), xs, W1, b1, W2, b2)
    out = _combine(ys, d1, d2, wa, wb)
    return out.reshape(batch, seq, hidden)


# speedup vs baseline: 1.0394x; 1.0394x over previous
import functools

import jax
import jax.numpy as jnp
from jax import lax
from jax.experimental import pallas as pl
from jax.experimental.pallas import tpu as pltpu
from jax.experimental.pallas import tpu_sc as plsc

N = 2048
H = 768
E = 8
CAP = 1024
XS_ROWS = E * CAP
TRASH = XS_ROWS
XS_TOT = XS_ROWS + 8

NC, NS = 2, 16
NW = NC * NS
TOK_PER = N // NW
LANES = 16


DLN = N // 128
WLN = N * LANES // 128


def _routing_body(tok_ref, wg_ref, d1_ref, d2_ref, wa_ref, wb_ref, cnt_ref):
    x = tok_ref[...]
    wg = wg_ref[...]
    logits = lax.dot_general(x, wg, (((1,), (1,)), ((), ())),
                             preferred_element_type=jnp.float32)
    m = jnp.max(logits, axis=1, keepdims=True)
    ex = jnp.exp(logits - m)
    probs = ex / jnp.sum(ex, axis=1, keepdims=True)

    lane = lax.broadcasted_iota(jnp.int32, (N, E), 1)
    m1 = jnp.max(probs, axis=1, keepdims=True)
    i1 = jnp.min(jnp.where(probs == m1, lane, E), axis=1, keepdims=True)
    probs2 = jnp.where(lane == i1, -1.0, probs)
    m2 = jnp.max(probs2, axis=1, keepdims=True)
    i2 = jnp.min(jnp.where(probs2 == m2, lane, E), axis=1, keepdims=True)

    oh1 = (lane == i1).astype(jnp.float32)
    oh2 = (lane == i2).astype(jnp.float32)
    s = oh1 + oh2

    kc = 256
    row_i = lax.broadcasted_iota(jnp.int32, (N, kc), 0)
    col_i = lax.broadcasted_iota(jnp.int32, (N, kc), 1)
    c = jnp.zeros((N, E), jnp.float32)
    for k in range(N // kc):
        mask = (col_i + k * kc < row_i).astype(jnp.float32)
        c = c + lax.dot_general(mask, s[k * kc:(k + 1) * kc, :],
                                (((1,), (0,)), ((), ())),
                                preferred_element_type=jnp.float32)

    counts = jnp.sum(s, axis=0, keepdims=True)
    counts_i = counts.astype(jnp.int32)
    kept = counts_i <= CAP

    r1 = jnp.sum(c * oh1, axis=1, keepdims=True)
    r2 = jnp.sum(c * oh2, axis=1, keepdims=True)
    keptf = kept.astype(jnp.float32)
    k1 = jnp.sum(keptf * oh1, axis=1, keepdims=True)
    k2 = jnp.sum(keptf * oh2, axis=1, keepdims=True)
    ro1 = i1.astype(jnp.float32) * float(CAP)
    ro2 = i2.astype(jnp.float32) * float(CAP)

    d1f = jnp.where(k1 > 0., ro1 + r1, float(TRASH))
    d2f = jnp.where(k2 > 0., ro2 + r2, float(TRASH))

    tt = lax.broadcasted_iota(jnp.int32, (N, 128), 0)
    ll = lax.broadcasted_iota(jnp.int32, (N, 128), 1)
    b_d = ((tt % 128) == ll).astype(jnp.float32)
    rr = lax.broadcasted_iota(jnp.int32, (DLN, N), 0)
    cc = lax.broadcasted_iota(jnp.int32, (DLN, N), 1)
    a_d = ((cc // 128) == rr).astype(jnp.float32)
    d1p = lax.dot_general(a_d, d1f * b_d, (((1,), (0,)), ((), ())),
                          precision=lax.Precision.HIGHEST,
                          preferred_element_type=jnp.float32)
    d2p = lax.dot_general(a_d, d2f * b_d, (((1,), (0,)), ((), ())),
                          precision=lax.Precision.HIGHEST,
                          preferred_element_type=jnp.float32)
    d1_ref[...] = (d1p + 0.5).astype(jnp.int32)
    d2_ref[...] = (d2p + 0.5).astype(jnp.int32)

    b_w = ((tt % 8) == (ll // LANES)).astype(jnp.float32)
    rr2 = lax.broadcasted_iota(jnp.int32, (WLN, N), 0)
    cc2 = lax.broadcasted_iota(jnp.int32, (WLN, N), 1)
    a_w = ((cc2 // 8) == rr2).astype(jnp.float32)
    wa_ref[...] = lax.dot_general(a_w, (m1 * k1) * b_w,
                                  (((1,), (0,)), ((), ())),
                                  precision=lax.Precision.HIGHEST,
                                  preferred_element_type=jnp.float32)
    wb_ref[...] = lax.dot_general(a_w, (m2 * k2) * b_w,
                                  (((1,), (0,)), ((), ())),
                                  precision=lax.Precision.HIGHEST,
                                  preferred_element_type=jnp.float32)
    cnt_ref[...] = counts_i


def _routing(flat_tokens, w_gate):
    return pl.pallas_call(
        _routing_body,
        out_shape=[
            jax.ShapeDtypeStruct((DLN, 128), jnp.int32),
            jax.ShapeDtypeStruct((DLN, 128), jnp.int32),
            jax.ShapeDtypeStruct((WLN, 128), jnp.float32),
            jax.ShapeDtypeStruct((WLN, 128), jnp.float32),
            jax.ShapeDtypeStruct((1, E), jnp.int32),
        ],
    )(flat_tokens, w_gate)


def _dispatch_body(tok_hbm, d1_hbm, d2_hbm, xs_hbm, d1_v, d2_v, rows_v,
                   sem_t, sem1, sem2):
    c = lax.axis_index("c")
    s = lax.axis_index("s")
    wid = s * NC + c
    base = wid * TOK_PER
    tok_cp = pltpu.async_copy(tok_hbm.at[pl.ds(base, TOK_PER)], rows_v, sem_t)
    pltpu.sync_copy(d1_hbm.at[pl.ds(base, TOK_PER)], d1_v)
    pltpu.sync_copy(d2_hbm.at[pl.ds(base, TOK_PER)], d2_v)
    tok_cp.wait()
    cp1 = pltpu.async_copy(rows_v, xs_hbm.at[d1_v], sem1)
    cp2 = pltpu.async_copy(rows_v, xs_hbm.at[d2_v], sem2)
    cp1.wait()
    cp2.wait()


def _dispatch(flat_tokens, d1, d2):
    mesh = plsc.VectorSubcoreMesh(core_axis_name="c", subcore_axis_name="s")
    return pl.kernel(
        _dispatch_body,
        out_type=jax.ShapeDtypeStruct((XS_TOT, H), jnp.float32),
        mesh=mesh,
        scratch_types=[
            pltpu.VMEM((TOK_PER,), jnp.int32),
            pltpu.VMEM((TOK_PER,), jnp.int32),
            pltpu.VMEM((TOK_PER, H), jnp.float32),
            pltpu.SemaphoreType.DMA,
            pltpu.SemaphoreType.DMA,
            pltpu.SemaphoreType.DMA,
        ],
    )(flat_tokens, d1, d2)


def _expert_body(cnt_ref, xs_ref, w1_ref, b1_ref, w2_ref, b2_ref, ys_ref):
    e = pl.program_id(0)
    cnt = cnt_ref[e]

    @pl.when((cnt > 0) & (cnt <= CAP))
    def _():
        x = xs_ref[...].astype(jnp.bfloat16)
        h = lax.dot_general(x, w1_ref[0].astype(jnp.bfloat16),
                            (((1,), (1,)), ((), ())),
                            preferred_element_type=jnp.float32)
        h = h + b1_ref[0]
        h = h * (1.0 / (1.0 + jnp.exp(-h)))
        y = lax.dot_general(h.astype(jnp.bfloat16),
                            w2_ref[0].astype(jnp.bfloat16),
                            (((1,), (1,)), ((), ())),
                            preferred_element_type=jnp.float32)
        ys_ref[...] = y + b2_ref[0]


def _experts(cnt, xs, w1, b1, w2, b2):
    def emap(e, cnt_s):
        return (e, 0, 0)

    grid_spec = pltpu.PrefetchScalarGridSpec(
        num_scalar_prefetch=1,
        grid=(E,),
        in_specs=[
            pl.BlockSpec((CAP, H), lambda e, cnt_s: (e, 0)),
            pl.BlockSpec((1, H, H), emap),
            pl.BlockSpec((1, 1, H), emap),
            pl.BlockSpec((1, H, H), emap),
            pl.BlockSpec((1, 1, H), emap),
        ],
        out_specs=pl.BlockSpec((CAP, H), lambda e, cnt_s: (e, 0)),
    )
    return pl.pallas_call(
        _expert_body,
        grid_spec=grid_spec,
        out_shape=jax.ShapeDtypeStruct((XS_ROWS, H), jnp.float32),
        compiler_params=pltpu.CompilerParams(
            dimension_semantics=("arbitrary",)),
    )(cnt, xs, w1, b1.reshape(E, 1, H), w2, b2.reshape(E, 1, H))


def _combine_body(ys_hbm, d1_hbm, d2_hbm, wa_hbm, wb_hbm, out_hbm,
                  d1_v, d2_v, wa_v, wb_v, r1_v, r2_v, sem1, sem2):
    c = lax.axis_index("c")
    s = lax.axis_index("s")
    wid = s * NC + c
    base = wid * TOK_PER
    pltpu.sync_copy(d1_hbm.at[pl.ds(base, TOK_PER)], d1_v)
    pltpu.sync_copy(d2_hbm.at[pl.ds(base, TOK_PER)], d2_v)
    pltpu.sync_copy(wa_hbm.at[pl.ds(base * LANES, TOK_PER * LANES)], wa_v)
    pltpu.sync_copy(wb_hbm.at[pl.ds(base * LANES, TOK_PER * LANES)], wb_v)
    for k in range(TOK_PER // LANES):
        sl = pl.ds(k * LANES, LANES)
        d1_v[sl] = jnp.minimum(d1_v[sl], XS_ROWS - 1)
        d2_v[sl] = jnp.minimum(d2_v[sl], XS_ROWS - 1)
    cp1 = pltpu.async_copy(ys_hbm.at[d1_v], r1_v, sem1)
    cp2 = pltpu.async_copy(ys_hbm.at[d2_v], r2_v, sem2)
    cp1.wait()
    cp2.wait()

    def row(j, _):
        wa = wa_v[pl.ds(j * LANES, LANES)]
        wb = wb_v[pl.ds(j * LANES, LANES)]
        zero = jnp.zeros((LANES,), jnp.float32)
        for ch in range(H // LANES):
            sl = pl.ds(ch * LANES, LANES)
            a = r1_v[j, sl]
            b = r2_v[j, sl]
            r1_v[j, sl] = (jnp.where(wa == 0.0, zero, a * wa)
                           + jnp.where(wb == 0.0, zero, b * wb))
        return 0

    lax.fori_loop(0, TOK_PER, row, 0)
    pltpu.sync_copy(r1_v, out_hbm.at[pl.ds(base, TOK_PER)])


def _combine(ys, d1, d2, wa, wb):
    mesh = plsc.VectorSubcoreMesh(core_axis_name="c", subcore_axis_name="s")
    return pl.kernel(
        _combine_body,
        out_type=jax.ShapeDtypeStruct((N, H), jnp.float32),
        mesh=mesh,
        scratch_types=[
            pltpu.VMEM((TOK_PER,), jnp.int32),
            pltpu.VMEM((TOK_PER,), jnp.int32),
            pltpu.VMEM((TOK_PER * LANES,), jnp.float32),
            pltpu.VMEM((TOK_PER * LANES,), jnp.float32),
            pltpu.VMEM((TOK_PER, H), jnp.float32),
            pltpu.VMEM((TOK_PER, H), jnp.float32),
            pltpu.SemaphoreType.DMA,
            pltpu.SemaphoreType.DMA,
        ],
    )(ys, d1, d2, wa, wb)


def kernel(tokens, W_gate, W1, b1, W2, b2):
    batch, seq, hidden = tokens.shape
    flat = tokens.reshape(batch * seq, hidden)
    d1, d2, wa, wb, cnt = _routing(flat, W_gate)
    d1 = d1.reshape(N)
    d2 = d2.reshape(N)
    xs = _dispatch(flat, d1, d2)
    ys = _experts(cnt.reshape(E), xs, W1, b1, W2, b2)
    out = _combine(ys, d1, d2, wa.reshape(N * LANES), wb.reshape(N * LANES))
    return out.reshape(batch, seq, hidden)
